# baseline (speedup 1.0000x reference)
import jax
import jax.numpy as jnp
from jax import lax
from jax.experimental import pallas as pl
from jax.experimental.pallas import tpu as pltpu
from jax.experimental.pallas import tpu_sc as plsc

NC = 2
NS = 16
NW = NC * NS
L = 16
B = 16384
BPW = B // NW
D = 32
GCH = 128
NG = BPW // GCH
MULT = 20.0


def _body(s_hbm, r_hbm, o_hbm, E_hbm, R_hbm, Et_hbm, Rht_hbm, Rtt_hbm,
          out_hbm,
          sidx, ridx, oidx, es, eo, est, eot, er, erht, ertt,
          outv, sem):
    wid = lax.axis_index("s") * NC + lax.axis_index("c")
    base = wid * BPW

    pltpu.sync_copy(s_hbm.at[pl.ds(base, BPW)], sidx)
    pltpu.sync_copy(r_hbm.at[pl.ds(base, BPW)], ridx)
    pltpu.sync_copy(o_hbm.at[pl.ds(base, BPW)], oidx)

    def remap(g, _):
        sl2 = pl.ds(g * L, L)
        for ref in (sidx, oidx):
            v = ref[sl2]
            ref[sl2] = ((lax.shift_right_logical(v, 11) * 2048)
                        + (v & 511) * 4
                        + lax.shift_right_logical(v & 2047, 9))
        return 0

    lax.fori_loop(0, BPW // L, remap, 0)

    copies = []
    for j in range(NG):
        sl = pl.ds(j * GCH, GCH)
        for tab, idx, dst in ((E_hbm, sidx, es), (E_hbm, oidx, eo),
                              (Et_hbm, sidx, est), (Et_hbm, oidx, eot),
                              (R_hbm, ridx, er), (Rht_hbm, ridx, erht),
                              (Rtt_hbm, ridx, ertt)):
            copies.append(pltpu.async_copy(
                tab.at[idx.at[sl]], dst.at[sl], sem))
    for c in copies:
        c.wait()

    lane = lax.iota(jnp.int32, L)
    cols = [(lane + d) % D for d in range(D)]

    def grp(g, _):
        zero = jnp.zeros((L,), jnp.float32)
        bacc, hacc, tacc = zero, zero, zero
        rows = g * L + lane
        for d in range(D):
            c = cols[d]
            bacc += (plsc.load_gather(es, [rows, c])
                     * plsc.load_gather(er, [rows, c])
                     * plsc.load_gather(eo, [rows, c]))
            hacc += (plsc.load_gather(est, [rows, c])
                     * plsc.load_gather(erht, [rows, c]))
            tacc += (plsc.load_gather(eot, [rows, c])
                     * plsc.load_gather(ertt, [rows, c]))
        sb = 1.0 / (1.0 + jnp.exp(-bacc))
        sh = 1.0 / (1.0 + jnp.exp(-hacc))
        st = 1.0 / (1.0 + jnp.exp(-tacc))
        outv[pl.ds(g * L, L)] = MULT * sb * sh * st
        return 0

    lax.fori_loop(0, BPW // L, grp, 0)

    pltpu.sync_copy(outv, out_hbm.at[pl.ds(base, BPW)])


QR = 128 // D
SUB = 512
EBLK = SUB * QR
NEB = (1000000 + EBLK - 1) // EBLK
NPAD = NEB * EBLK


def _tc_transpose_body(x_ref, out_ref):
    x = x_ref[...]
    eye = (lax.broadcasted_iota(jnp.int32, (D, D), 0)
           == lax.broadcasted_iota(jnp.int32, (D, D), 1)).astype(jnp.float32)
    for j in range(QR):
        xs = x[:, SUB * j:SUB * (j + 1)]
        out_ref[:, D * j:D * (j + 1)] = lax.dot_general(
            xs, eye, (((0,), (0,)), ((), ())),
            preferred_element_type=jnp.float32)


def _to_row_major_permuted(t_tr):
    flat = pl.pallas_call(
        _tc_transpose_body,
        grid=(NEB,),
        in_specs=[pl.BlockSpec((D, EBLK), lambda i: (0, i))],
        out_specs=pl.BlockSpec((SUB, 128), lambda i: (i, 0)),
        out_shape=jax.ShapeDtypeStruct((NPAD * D // 128, 128), jnp.float32),
    )(t_tr)
    return flat.reshape(NPAD, D)


def kernel(s, r, o, E, R, E_t, R_ht, R_tt):
    mesh = plsc.VectorSubcoreMesh(
        core_axis_name="c", subcore_axis_name="s",
        num_cores=NC, num_subcores=NS)
    f = pl.kernel(
        _body,
        out_type=jax.ShapeDtypeStruct((B,), jnp.float32),
        mesh=mesh,
        compiler_params=pltpu.CompilerParams(
            needs_layout_passes=False, use_tc_tiling_on_sc=False),
        scratch_types=[
            pltpu.VMEM((BPW,), jnp.int32),
            pltpu.VMEM((BPW,), jnp.int32),
            pltpu.VMEM((BPW,), jnp.int32),
            pltpu.VMEM((BPW, D), jnp.float32),
            pltpu.VMEM((BPW, D), jnp.float32),
            pltpu.VMEM((BPW, D), jnp.float32),
            pltpu.VMEM((BPW, D), jnp.float32),
            pltpu.VMEM((BPW, D), jnp.float32),
            pltpu.VMEM((BPW, D), jnp.float32),
            pltpu.VMEM((BPW, D), jnp.float32),
            pltpu.VMEM((BPW,), jnp.float32),
            pltpu.SemaphoreType.DMA,
        ],
    )
    return f(s.astype(jnp.int32), r.astype(jnp.int32), o.astype(jnp.int32),
             _to_row_major_permuted(E.T), R,
             _to_row_major_permuted(E_t.T), R_ht, R_tt)


# speedup vs baseline: 1.6039x; 1.6039x over previous
import jax
import jax.numpy as jnp
from jax import lax
from jax.experimental import pallas as pl
from jax.experimental.pallas import tpu as pltpu
from jax.experimental.pallas import tpu_sc as plsc

NC = 2
NS = 16
NW = NC * NS
L = 16
B = 16384
BPW = B // NW
D = 32
GCH = 128
NG = BPW // GCH
MULT = 20.0


def _body(s_hbm, r_hbm, o_hbm, E_hbm, R_hbm, Et_hbm, Rht_hbm, Rtt_hbm,
          out_hbm,
          sidx, ridx, oidx, es, eo, est, eot, er, erht, ertt,
          outv, sem):
    wid = lax.axis_index("s") * NC + lax.axis_index("c")
    base = wid * BPW

    pltpu.sync_copy(s_hbm.at[pl.ds(base, BPW)], sidx)
    pltpu.sync_copy(r_hbm.at[pl.ds(base, BPW)], ridx)
    pltpu.sync_copy(o_hbm.at[pl.ds(base, BPW)], oidx)

    def remap(g, _):
        sl2 = pl.ds(g * L, L)
        for ref in (sidx, oidx):
            v = ref[sl2]
            m = v % EBLK
            ref[sl2] = (v - m) + (m % SUB) * QR + m // SUB
        return 0

    lax.fori_loop(0, BPW // L, remap, 0)

    copies = []
    for j in range(NG):
        sl = pl.ds(j * GCH, GCH)
        for tab, idx, dst in ((E_hbm, sidx, es), (E_hbm, oidx, eo),
                              (Et_hbm, sidx, est), (Et_hbm, oidx, eot),
                              (R_hbm, ridx, er), (Rht_hbm, ridx, erht),
                              (Rtt_hbm, ridx, ertt)):
            copies.append(pltpu.async_copy(
                tab.at[idx.at[sl]], dst.at[sl], sem))
    for c in copies:
        c.wait()

    lane = lax.iota(jnp.int32, L)
    cols = [(lane + d) % D for d in range(D)]

    def grp(g, _):
        zero = jnp.zeros((L,), jnp.float32)
        bacc, hacc, tacc = zero, zero, zero
        rows = g * L + lane
        for d in range(D):
            c = cols[d]
            bacc += (plsc.load_gather(es, [rows, c])
                     * plsc.load_gather(er, [rows, c])
                     * plsc.load_gather(eo, [rows, c]))
            hacc += (plsc.load_gather(est, [rows, c])
                     * plsc.load_gather(erht, [rows, c]))
            tacc += (plsc.load_gather(eot, [rows, c])
                     * plsc.load_gather(ertt, [rows, c]))
        sb = 1.0 / (1.0 + jnp.exp(-bacc))
        sh = 1.0 / (1.0 + jnp.exp(-hacc))
        st = 1.0 / (1.0 + jnp.exp(-tacc))
        outv[pl.ds(g * L, L)] = MULT * sb * sh * st
        return 0

    lax.fori_loop(0, BPW // L, grp, 0)

    pltpu.sync_copy(outv, out_hbm.at[pl.ds(base, BPW)])


QR = 128 // D
SUB = 2048
EBLK = SUB * QR
NEB = (1000000 + EBLK - 1) // EBLK
NPAD = NEB * EBLK


def _tc_transpose_body(x_ref, out_ref):
    x = x_ref[...]
    eye = (lax.broadcasted_iota(jnp.int32, (D, D), 0)
           == lax.broadcasted_iota(jnp.int32, (D, D), 1)).astype(jnp.float32)
    for j in range(QR):
        xs = x[:, SUB * j:SUB * (j + 1)]
        out_ref[:, D * j:D * (j + 1)] = lax.dot_general(
            xs, eye, (((0,), (0,)), ((), ())),
            preferred_element_type=jnp.float32)


def _to_row_major_permuted(t_tr):
    flat = pl.pallas_call(
        _tc_transpose_body,
        grid=(NEB,),
        in_specs=[pl.BlockSpec((D, EBLK), lambda i: (0, i))],
        out_specs=pl.BlockSpec((SUB, 128), lambda i: (i, 0)),
        out_shape=jax.ShapeDtypeStruct((NPAD * D // 128, 128), jnp.float32),
    )(t_tr)
    return flat.reshape(NPAD, D)


def kernel(s, r, o, E, R, E_t, R_ht, R_tt):
    mesh = plsc.VectorSubcoreMesh(
        core_axis_name="c", subcore_axis_name="s",
        num_cores=NC, num_subcores=NS)
    f = pl.kernel(
        _body,
        out_type=jax.ShapeDtypeStruct((B,), jnp.float32),
        mesh=mesh,
        compiler_params=pltpu.CompilerParams(
            needs_layout_passes=False, use_tc_tiling_on_sc=False),
        scratch_types=[
            pltpu.VMEM((BPW,), jnp.int32),
            pltpu.VMEM((BPW,), jnp.int32),
            pltpu.VMEM((BPW,), jnp.int32),
            pltpu.VMEM((BPW, D), jnp.float32),
            pltpu.VMEM((BPW, D), jnp.float32),
            pltpu.VMEM((BPW, D), jnp.float32),
            pltpu.VMEM((BPW, D), jnp.float32),
            pltpu.VMEM((BPW, D), jnp.float32),
            pltpu.VMEM((BPW, D), jnp.float32),
            pltpu.VMEM((BPW, D), jnp.float32),
            pltpu.VMEM((BPW,), jnp.float32),
            pltpu.SemaphoreType.DMA,
        ],
    )
    return f(s.astype(jnp.int32), r.astype(jnp.int32), o.astype(jnp.int32),
             _to_row_major_permuted(E.T), R,
             _to_row_major_permuted(E_t.T), R_ht, R_tt)


# speedup vs baseline: 1.6379x; 1.0212x over previous
import jax
import jax.numpy as jnp
from jax import lax
from jax.experimental import pallas as pl
from jax.experimental.pallas import tpu as pltpu
from jax.experimental.pallas import tpu_sc as plsc

NC = 2
NS = 16
NW = NC * NS
L = 16
B = 16384
BPW = B // NW
D = 32
GCH = 128
NG = BPW // GCH
MULT = 20.0


def _body(s_hbm, r_hbm, o_hbm, E_hbm, R_hbm, Et_hbm, Rht_hbm, Rtt_hbm,
          out_hbm,
          sidx, ridx, oidx, es, eo, est, eot, er, erht, ertt,
          outv, sem):
    wid = lax.axis_index("s") * NC + lax.axis_index("c")
    base = wid * BPW

    pltpu.sync_copy(s_hbm.at[pl.ds(base, BPW)], sidx)
    pltpu.sync_copy(r_hbm.at[pl.ds(base, BPW)], ridx)
    pltpu.sync_copy(o_hbm.at[pl.ds(base, BPW)], oidx)

    def remap(g, _):
        sl2 = pl.ds(g * L, L)
        for ref in (sidx, oidx):
            v = ref[sl2]
            m = v % EBLK
            ref[sl2] = (v - m) + (m % SUB) * QR + m // SUB
        return 0

    lax.fori_loop(0, BPW // L, remap, 0)

    copies = []
    for j in range(NG):
        sl = pl.ds(j * GCH, GCH)
        for tab, idx, dst in ((E_hbm, sidx, es), (E_hbm, oidx, eo),
                              (Et_hbm, sidx, est), (Et_hbm, oidx, eot),
                              (R_hbm, ridx, er), (Rht_hbm, ridx, erht),
                              (Rtt_hbm, ridx, ertt)):
            copies.append(pltpu.async_copy(
                tab.at[idx.at[sl]], dst.at[sl], sem))
    for c in copies:
        c.wait()

    lane = lax.iota(jnp.int32, L)
    cols = [(lane + d) % D for d in range(D)]

    def grp(g, _):
        zero = jnp.zeros((L,), jnp.float32)
        bacc, hacc, tacc = zero, zero, zero
        rows = g * L + lane
        for d in range(D):
            c = cols[d]
            bacc += (plsc.load_gather(es, [rows, c])
                     * plsc.load_gather(er, [rows, c])
                     * plsc.load_gather(eo, [rows, c]))
            hacc += (plsc.load_gather(est, [rows, c])
                     * plsc.load_gather(erht, [rows, c]))
            tacc += (plsc.load_gather(eot, [rows, c])
                     * plsc.load_gather(ertt, [rows, c]))
        sb = 1.0 / (1.0 + jnp.exp(-bacc))
        sh = 1.0 / (1.0 + jnp.exp(-hacc))
        st = 1.0 / (1.0 + jnp.exp(-tacc))
        outv[pl.ds(g * L, L)] = MULT * sb * sh * st
        return 0

    lax.fori_loop(0, BPW // L, grp, 0)

    pltpu.sync_copy(outv, out_hbm.at[pl.ds(base, BPW)])


QR = 128 // D
SUB = 8192
EBLK = SUB * QR
NEB = (1000000 + EBLK - 1) // EBLK
NPAD = NEB * EBLK


def _tc_transpose_body(x_ref, out_ref):
    x = x_ref[...]
    eye = (lax.broadcasted_iota(jnp.int32, (D, D), 0)
           == lax.broadcasted_iota(jnp.int32, (D, D), 1)).astype(jnp.float32)
    for j in range(QR):
        xs = x[:, SUB * j:SUB * (j + 1)]
        out_ref[:, D * j:D * (j + 1)] = lax.dot_general(
            xs, eye, (((0,), (0,)), ((), ())),
            preferred_element_type=jnp.float32)


def _to_row_major_permuted(t_tr):
    flat = pl.pallas_call(
        _tc_transpose_body,
        grid=(NEB,),
        in_specs=[pl.BlockSpec((D, EBLK), lambda i: (0, i))],
        out_specs=pl.BlockSpec((SUB, 128), lambda i: (i, 0)),
        out_shape=jax.ShapeDtypeStruct((NPAD * D // 128, 128), jnp.float32),
    )(t_tr)
    return flat.reshape(NPAD, D)


def kernel(s, r, o, E, R, E_t, R_ht, R_tt):
    mesh = plsc.VectorSubcoreMesh(
        core_axis_name="c", subcore_axis_name="s",
        num_cores=NC, num_subcores=NS)
    f = pl.kernel(
        _body,
        out_type=jax.ShapeDtypeStruct((B,), jnp.float32),
        mesh=mesh,
        compiler_params=pltpu.CompilerParams(
            needs_layout_passes=False, use_tc_tiling_on_sc=False),
        scratch_types=[
            pltpu.VMEM((BPW,), jnp.int32),
            pltpu.VMEM((BPW,), jnp.int32),
            pltpu.VMEM((BPW,), jnp.int32),
            pltpu.VMEM((BPW, D), jnp.float32),
            pltpu.VMEM((BPW, D), jnp.float32),
            pltpu.VMEM((BPW, D), jnp.float32),
            pltpu.VMEM((BPW, D), jnp.float32),
            pltpu.VMEM((BPW, D), jnp.float32),
            pltpu.VMEM((BPW, D), jnp.float32),
            pltpu.VMEM((BPW, D), jnp.float32),
            pltpu.VMEM((BPW,), jnp.float32),
            pltpu.SemaphoreType.DMA,
        ],
    )
    return f(s.astype(jnp.int32), r.astype(jnp.int32), o.astype(jnp.int32),
             _to_row_major_permuted(E.T), R,
             _to_row_major_permuted(E_t.T), R_ht, R_tt)


# speedup vs baseline: 1.6380x; 1.0001x over previous
import jax
import jax.numpy as jnp
from jax import lax
from jax.experimental import pallas as pl
from jax.experimental.pallas import tpu as pltpu
from jax.experimental.pallas import tpu_sc as plsc

NC = 2
NS = 16
NW = NC * NS
L = 16
B = 16384
BPW = B // NW
D = 32
GCH = 128
NG = BPW // GCH
MULT = 20.0


def _body(s_hbm, r_hbm, o_hbm, E_hbm, R_hbm, Et_hbm, Rht_hbm, Rtt_hbm,
          out_hbm,
          sidx, ridx, oidx, es, eo, est, eot, er, erht, ertt,
          outv, sem):
    wid = lax.axis_index("s") * NC + lax.axis_index("c")
    base = wid * BPW

    pltpu.sync_copy(s_hbm.at[pl.ds(base, BPW)], sidx)
    pltpu.sync_copy(r_hbm.at[pl.ds(base, BPW)], ridx)
    pltpu.sync_copy(o_hbm.at[pl.ds(base, BPW)], oidx)

    def remap(g, _):
        sl2 = pl.ds(g * L, L)
        for ref in (sidx, oidx):
            v = ref[sl2]
            m = v % EBLK
            ref[sl2] = (v - m) + (m % SUB) * QR + m // SUB
        return 0

    lax.fori_loop(0, BPW // L, remap, 0)

    copies = []
    for j in range(NG):
        sl = pl.ds(j * GCH, GCH)
        for tab, idx, dst in ((E_hbm, sidx, es), (E_hbm, oidx, eo),
                              (Et_hbm, sidx, est), (Et_hbm, oidx, eot),
                              (R_hbm, ridx, er), (Rht_hbm, ridx, erht),
                              (Rtt_hbm, ridx, ertt)):
            copies.append(pltpu.async_copy(
                tab.at[idx.at[sl]], dst.at[sl], sem))
    for c in copies:
        c.wait()

    lane = lax.iota(jnp.int32, L)
    cols = [(lane + d) % D for d in range(D)]

    def grp(g, _):
        zero = jnp.zeros((L,), jnp.float32)
        bacc, hacc, tacc = zero, zero, zero
        rows = g * L + lane
        for d in range(D):
            c = cols[d]
            bacc += (plsc.load_gather(es, [rows, c])
                     * plsc.load_gather(er, [rows, c])
                     * plsc.load_gather(eo, [rows, c]))
            hacc += (plsc.load_gather(est, [rows, c])
                     * plsc.load_gather(erht, [rows, c]))
            tacc += (plsc.load_gather(eot, [rows, c])
                     * plsc.load_gather(ertt, [rows, c]))
        sb = 1.0 / (1.0 + jnp.exp(-bacc))
        sh = 1.0 / (1.0 + jnp.exp(-hacc))
        st = 1.0 / (1.0 + jnp.exp(-tacc))
        outv[pl.ds(g * L, L)] = MULT * sb * sh * st
        return 0

    lax.fori_loop(0, BPW // L, grp, 0)

    pltpu.sync_copy(outv, out_hbm.at[pl.ds(base, BPW)])


QR = 128 // D
SUB = 8192
EBLK = SUB * QR
NEB = (1000000 + EBLK - 1) // EBLK
NPAD = NEB * EBLK


def _tc_transpose_body(x_ref, out_ref):
    x = x_ref[...]
    eye = (lax.broadcasted_iota(jnp.int32, (D, D), 0)
           == lax.broadcasted_iota(jnp.int32, (D, D), 1)).astype(jnp.float32)
    for j in range(QR):
        xs = x[:, SUB * j:SUB * (j + 1)]
        out_ref[:, D * j:D * (j + 1)] = lax.dot_general(
            xs, eye, (((0,), (0,)), ((), ())),
            preferred_element_type=jnp.float32)


def _to_row_major_permuted(t_tr):
    flat = pl.pallas_call(
        _tc_transpose_body,
        grid=(NEB,),
        compiler_params=pltpu.CompilerParams(
            fuse_transposed_lhs_in_matmul=True),
        in_specs=[pl.BlockSpec((D, EBLK), lambda i: (0, i))],
        out_specs=pl.BlockSpec((SUB, 128), lambda i: (i, 0)),
        out_shape=jax.ShapeDtypeStruct((NPAD * D // 128, 128), jnp.float32),
    )(t_tr)
    return flat.reshape(NPAD, D)


def kernel(s, r, o, E, R, E_t, R_ht, R_tt):
    mesh = plsc.VectorSubcoreMesh(
        core_axis_name="c", subcore_axis_name="s",
        num_cores=NC, num_subcores=NS)
    f = pl.kernel(
        _body,
        out_type=jax.ShapeDtypeStruct((B,), jnp.float32),
        mesh=mesh,
        compiler_params=pltpu.CompilerParams(
            needs_layout_passes=False, use_tc_tiling_on_sc=False),
        scratch_types=[
            pltpu.VMEM((BPW,), jnp.int32),
            pltpu.VMEM((BPW,), jnp.int32),
            pltpu.VMEM((BPW,), jnp.int32),
            pltpu.VMEM((BPW, D), jnp.float32),
            pltpu.VMEM((BPW, D), jnp.float32),
            pltpu.VMEM((BPW, D), jnp.float32),
            pltpu.VMEM((BPW, D), jnp.float32),
            pltpu.VMEM((BPW, D), jnp.float32),
            pltpu.VMEM((BPW, D), jnp.float32),
            pltpu.VMEM((BPW, D), jnp.float32),
            pltpu.VMEM((BPW,), jnp.float32),
            pltpu.SemaphoreType.DMA,
        ],
    )
    return f(s.astype(jnp.int32), r.astype(jnp.int32), o.astype(jnp.int32),
             _to_row_major_permuted(E.T), R,
             _to_row_major_permuted(E_t.T), R_ht, R_tt)
